# baseline (speedup 1.0000x reference)
import functools

import jax
import jax.numpy as jnp
from jax import lax
from jax.experimental import pallas as pl
from jax.experimental.pallas import tpu as pltpu
from jax.experimental.pallas import tpu_sc as plsc

_CHUNK = 128


@functools.lru_cache(maxsize=None)
def _make_route_gather(B, D):
    info = plsc.get_sparse_core_info()
    NC, NS = 1, info.num_subcores
    NW = NC * NS
    assert B % (8 * NW) == 0 and D % info.num_lanes == 0
    b_per_w = B // NW
    assert b_per_w % _CHUNK == 0
    n_chunks = b_per_w // _CHUNK
    mesh = plsc.VectorSubcoreMesh(
        core_axis_name="c", subcore_axis_name="s", num_cores=NC
    )

    @functools.partial(
        pl.kernel,
        mesh=mesh,
        out_type=jax.ShapeDtypeStruct((B, D), jnp.float32),
        scratch_types=[
            pltpu.VMEM((b_per_w,), jnp.int32),
            pltpu.VMEM((b_per_w, D), jnp.float32),
            pltpu.SemaphoreType.DMA,
            pltpu.SemaphoreType.DMA,
        ],
        compiler_params=pltpu.CompilerParams(use_tc_tiling_on_sc=False),
    )
    def gather_rows(table_hbm, idx_hbm, out_hbm, idx_v, rows_v, sem_g, sem_s):
        wid = lax.axis_index("s") * NC + lax.axis_index("c")
        base = wid * b_per_w
        pltpu.sync_copy(idx_hbm.at[pl.ds(base, b_per_w)], idx_v)
        gathers = []
        for j in range(n_chunks):
            sl = pl.ds(j * _CHUNK, _CHUNK)
            gathers.append(
                pltpu.async_copy(table_hbm.at[idx_v.at[sl]], rows_v.at[sl], sem_g)
            )
        stores = []
        for j in range(n_chunks):
            gathers[j].wait()
            sl = pl.ds(j * _CHUNK, _CHUNK)
            stores.append(
                pltpu.async_copy(
                    rows_v.at[sl], out_hbm.at[pl.ds(base + j * _CHUNK, _CHUNK)], sem_s
                )
            )
        for st in stores:
            st.wait()

    return gather_rows


def kernel(layer_input, labels, temperature, balance_coefficient, route_matrix):
    B = labels.shape[0]
    D = route_matrix.shape[1]
    gather = _make_route_gather(B, D)
    return gather(route_matrix, labels.astype(jnp.int32))
